# baseline (speedup 1.0000x reference)
import jax
import jax.numpy as jnp
from jax.experimental import pallas as pl
from jax.experimental.pallas import tpu as pltpu

_HIGHEST = jax.lax.Precision.HIGHEST


def _make_fused_kernel(ppd, nb):
    def _fused_kernel(x_ref, y_ref, pmat_ref, w1f3_ref, b1f_ref, wx_ref,
                      bx_ref, wy_ref, by_ref, o_ref):
        C = x_ref.shape[1]
        for i in range(nb):
            xv = x_ref[i]
            yv = y_ref[i]
            u = xv + yv
            pooled = jnp.dot(u, pmat_ref[...], precision=_HIGHEST,
                             preferred_element_type=jnp.float32)
            acc = pooled[:, 0:1] * w1f3_ref[0]
            for p in range(1, ppd):
                acc = acc + pooled[:, p:p + 1] * w1f3_ref[p]
            z = jnp.sum(acc, axis=0, keepdims=True) + b1f_ref[...]
            z = jnp.maximum(z, 0.0)
            zx = jnp.dot(z, wx_ref[...], precision=_HIGHEST,
                         preferred_element_type=jnp.float32) + bx_ref[...]
            zy = jnp.dot(z, wy_ref[...], precision=_HIGHEST,
                         preferred_element_type=jnp.float32) + by_ref[...]
            m = jnp.maximum(zx, zy)
            ex = jnp.exp(zx - m)
            ey = jnp.exp(zy - m)
            wxr = ex / (ex + ey)
            rows = jax.lax.broadcasted_iota(jnp.int32, (C, C), 0)
            cols = jax.lax.broadcasted_iota(jnp.int32, (C, C), 1)
            wxc = jnp.sum(jnp.where(rows == cols, wxr, 0.0), axis=1,
                          keepdims=True)
            wyc = 1.0 - wxc
            o_ref[i] = (xv * wxc + yv * wyc).astype(o_ref.dtype)

    return _fused_kernel


def kernel(x, y, pmat, w1f, b1f, wx, bx, wy, by):
    B, C, H, W = x.shape
    HW = H * W
    D = w1f.shape[1]
    L = pmat.shape[1]
    ppd = w1f.shape[0] // C

    nb = 2 if B % 2 == 0 else 1

    xf = x.reshape(B, C, HW)
    yf = y.reshape(B, C, HW)
    w1f3 = w1f.reshape(C, ppd, D).transpose(1, 0, 2)
    b1f2 = b1f.reshape(1, D)
    bx2 = bx.reshape(1, C)
    by2 = by.reshape(1, C)

    out = pl.pallas_call(
        _make_fused_kernel(ppd, nb),
        out_shape=jax.ShapeDtypeStruct((B, C, HW), x.dtype),
        grid=(B // nb,),
        in_specs=[
            pl.BlockSpec((nb, C, HW), lambda b: (b, 0, 0)),
            pl.BlockSpec((nb, C, HW), lambda b: (b, 0, 0)),
            pl.BlockSpec((HW, L), lambda b: (0, 0)),
            pl.BlockSpec((ppd, C, D), lambda b: (0, 0, 0)),
            pl.BlockSpec((1, D), lambda b: (0, 0)),
            pl.BlockSpec((D, C), lambda b: (0, 0)),
            pl.BlockSpec((1, C), lambda b: (0, 0)),
            pl.BlockSpec((D, C), lambda b: (0, 0)),
            pl.BlockSpec((1, C), lambda b: (0, 0)),
        ],
        out_specs=pl.BlockSpec((nb, C, HW), lambda b: (b, 0, 0)),
        compiler_params=pltpu.CompilerParams(
            dimension_semantics=("parallel",),
            vmem_limit_bytes=48 << 20),
    )(xf, yf, pmat, w1f3, b1f2, wx, bx2, wy, by2)

    return out.reshape(B, C, H, W), None, y


# speedup vs baseline: 1.1736x; 1.1736x over previous
import jax
import jax.numpy as jnp
from jax.experimental import pallas as pl
from jax.experimental.pallas import tpu as pltpu

_HIGHEST = jax.lax.Precision.HIGHEST


def _make_fused_kernel(ppd, nb):
    def _fused_kernel(x_ref, y_ref, pmat_ref, w1f3_ref, b1f_ref, wx_ref,
                      bx_ref, wy_ref, by_ref, o_ref):
        C = x_ref.shape[1]
        for i in range(nb):
            xv = x_ref[i]
            yv = y_ref[i]
            u = xv + yv
            pooled = jnp.dot(u, pmat_ref[...],
                             preferred_element_type=jnp.float32)
            acc = pooled[:, 0:1] * w1f3_ref[0]
            for p in range(1, ppd):
                acc = acc + pooled[:, p:p + 1] * w1f3_ref[p]
            z = jnp.sum(acc, axis=0, keepdims=True) + b1f_ref[...]
            z = jnp.maximum(z, 0.0)
            zx = jnp.dot(z, wx_ref[...], precision=_HIGHEST,
                         preferred_element_type=jnp.float32) + bx_ref[...]
            zy = jnp.dot(z, wy_ref[...], precision=_HIGHEST,
                         preferred_element_type=jnp.float32) + by_ref[...]
            m = jnp.maximum(zx, zy)
            ex = jnp.exp(zx - m)
            ey = jnp.exp(zy - m)
            wxr = ex / (ex + ey)
            rows = jax.lax.broadcasted_iota(jnp.int32, (C, C), 0)
            cols = jax.lax.broadcasted_iota(jnp.int32, (C, C), 1)
            wxc = jnp.sum(jnp.where(rows == cols, wxr, 0.0), axis=1,
                          keepdims=True)
            wyc = 1.0 - wxc
            o_ref[i] = (xv * wxc + yv * wyc).astype(o_ref.dtype)

    return _fused_kernel


def kernel(x, y, pmat, w1f, b1f, wx, bx, wy, by):
    B, C, H, W = x.shape
    HW = H * W
    D = w1f.shape[1]
    L = pmat.shape[1]
    ppd = w1f.shape[0] // C

    nb = 1

    xf = x.reshape(B, C, HW)
    yf = y.reshape(B, C, HW)
    w1f3 = w1f.reshape(C, ppd, D).transpose(1, 0, 2)
    b1f2 = b1f.reshape(1, D)
    bx2 = bx.reshape(1, C)
    by2 = by.reshape(1, C)

    out = pl.pallas_call(
        _make_fused_kernel(ppd, nb),
        out_shape=jax.ShapeDtypeStruct((B, C, HW), x.dtype),
        grid=(B // nb,),
        in_specs=[
            pl.BlockSpec((nb, C, HW), lambda b: (b, 0, 0)),
            pl.BlockSpec((nb, C, HW), lambda b: (b, 0, 0)),
            pl.BlockSpec((HW, L), lambda b: (0, 0)),
            pl.BlockSpec((ppd, C, D), lambda b: (0, 0, 0)),
            pl.BlockSpec((1, D), lambda b: (0, 0)),
            pl.BlockSpec((D, C), lambda b: (0, 0)),
            pl.BlockSpec((1, C), lambda b: (0, 0)),
            pl.BlockSpec((D, C), lambda b: (0, 0)),
            pl.BlockSpec((1, C), lambda b: (0, 0)),
        ],
        out_specs=pl.BlockSpec((nb, C, HW), lambda b: (b, 0, 0)),
        compiler_params=pltpu.CompilerParams(
            dimension_semantics=("parallel",),
            vmem_limit_bytes=48 << 20),
    )(xf, yf, pmat, w1f3, b1f2, wx, bx2, wy, by2)

    return out.reshape(B, C, H, W), None, y
